# baseline (speedup 1.0000x reference)
import functools

import jax
import jax.numpy as jnp
from jax import lax
from jax.experimental import pallas as pl
from jax.experimental.pallas import tpu as pltpu
from jax.experimental.pallas import tpu_sc as plsc

_D = 1024
_FF = 512
_E = 8
_K = 2
_T = 2048
_N = _T * _K
_B = 128
_NP = _N + _E * _B
_NBP = _NP // _B
_NC = 2
_NS = 16
_NW = _NC * _NS


def _router_body(x_ref, rw_ref, rb_ref, idx_ref, w_ref, xc_ref):
    x = x_ref[...]
    rw = rw_ref[...]
    t = x.shape[0]
    logits = lax.dot_general(x, rw, (((1,), (1,)), ((), ())),
                             preferred_element_type=jnp.float32)
    scores = jax.nn.sigmoid(logits)
    sc = scores + rb_ref[...]
    idx8 = lax.broadcasted_iota(jnp.int32, (t, _E), 1)
    m1 = jnp.max(sc, axis=1, keepdims=True)
    i1 = jnp.min(jnp.where(sc >= m1, idx8, _E), axis=1, keepdims=True)
    w1 = jnp.sum(jnp.where(idx8 == i1, scores, 0.0), axis=1, keepdims=True)
    sc2 = jnp.where(idx8 == i1, -jnp.inf, sc)
    m2 = jnp.max(sc2, axis=1, keepdims=True)
    i2 = jnp.min(jnp.where(sc2 >= m2, idx8, _E), axis=1, keepdims=True)
    w2 = jnp.sum(jnp.where(idx8 == i2, scores, 0.0), axis=1, keepdims=True)
    denom = w1 + w2 + 1e-20
    idx_ref[...] = jnp.concatenate([i1, i2], axis=1)
    w_ref[...] = jnp.concatenate([w1 / denom, w2 / denom], axis=1)
    xc_ref[...] = x


def _sc_gather_body(total_rows, x_hbm, tok_hbm, xs_hbm,
                    idx0_v, idx1_v, idx2_v, idx3_v, buf0, buf1, sem0, sem1):
    wid = lax.axis_index("s") * _NC + lax.axis_index("c")
    rows_per_w = total_rows // _NW
    chunk = rows_per_w // 4
    base = wid * rows_per_w
    idx = (idx0_v, idx1_v, idx2_v, idx3_v)
    for j in range(4):
        pltpu.sync_copy(tok_hbm.at[pl.ds(base + j * chunk, chunk)], idx[j])
    cp0 = pltpu.async_copy(x_hbm.at[idx0_v], buf0, sem0)
    cp1 = pltpu.async_copy(x_hbm.at[idx1_v], buf1, sem1)
    cp0.wait()
    pltpu.sync_copy(buf0, xs_hbm.at[pl.ds(base, chunk)])
    cp2 = pltpu.async_copy(x_hbm.at[idx2_v], buf0, sem0)
    cp1.wait()
    pltpu.sync_copy(buf1, xs_hbm.at[pl.ds(base + chunk, chunk)])
    cp3 = pltpu.async_copy(x_hbm.at[idx3_v], buf1, sem1)
    cp2.wait()
    pltpu.sync_copy(buf0, xs_hbm.at[pl.ds(base + 2 * chunk, chunk)])
    cp3.wait()
    pltpu.sync_copy(buf1, xs_hbm.at[pl.ds(base + 3 * chunk, chunk)])


def _make_sc_gather(total_rows, out_rows_cols):
    chunk = total_rows // _NW // 4
    return functools.partial(
        pl.kernel,
        mesh=plsc.VectorSubcoreMesh(core_axis_name="c", subcore_axis_name="s"),
        out_type=jax.ShapeDtypeStruct(out_rows_cols, jnp.float32),
        scratch_types=[
            pltpu.VMEM((chunk,), jnp.int32),
            pltpu.VMEM((chunk,), jnp.int32),
            pltpu.VMEM((chunk,), jnp.int32),
            pltpu.VMEM((chunk,), jnp.int32),
            pltpu.VMEM((chunk, _D), jnp.float32),
            pltpu.VMEM((chunk, _D), jnp.float32),
            pltpu.SemaphoreType.DMA,
            pltpu.SemaphoreType.DMA,
        ],
    )(functools.partial(_sc_gather_body, total_rows))


def _gemm_body(be_ref, wsc_ref, xs_ref, wg_ref, wu_ref, wd_ref, ys_ref,
               wg_s, wu_s, wd_s):
    g = pl.program_id(0)
    prev = jnp.where(g == 0, -1, be_ref[jnp.maximum(g - 1, 0)])

    @pl.when(be_ref[g] != prev)
    def _():
        wg_s[...] = wg_ref[0].astype(jnp.bfloat16)
        wu_s[...] = wu_ref[0].astype(jnp.bfloat16)
        wd_s[...] = wd_ref[0].astype(jnp.bfloat16)

    xb = xs_ref[...].astype(jnp.bfloat16)
    gg = lax.dot_general(xb, wg_s[...], (((1,), (1,)), ((), ())),
                         preferred_element_type=jnp.float32)
    uu = lax.dot_general(xb, wu_s[...], (((1,), (1,)), ((), ())),
                         preferred_element_type=jnp.float32)
    h = ((gg * jax.nn.sigmoid(gg)) * uu).astype(jnp.bfloat16)
    y = lax.dot_general(h, wd_s[...], (((1,), (1,)), ((), ())),
                        preferred_element_type=jnp.float32)
    ys_ref[...] = y * wsc_ref[0, 0, :][:, None]


def _shared_body(x_ref, sg_ref, su_ref, sd_ref, ya_ref, yb_ref, out_ref,
                 sg_s, su_s, sd_s):
    @pl.when(pl.program_id(0) == 0)
    def _():
        sg_s[...] = sg_ref[...].astype(jnp.bfloat16)
        su_s[...] = su_ref[...].astype(jnp.bfloat16)
        sd_s[...] = sd_ref[...].astype(jnp.bfloat16)

    xb = x_ref[...].astype(jnp.bfloat16)
    g = lax.dot_general(xb, sg_s[...], (((1,), (1,)), ((), ())),
                        preferred_element_type=jnp.float32)
    u = lax.dot_general(xb, su_s[...], (((1,), (1,)), ((), ())),
                        preferred_element_type=jnp.float32)
    h = ((g * jax.nn.sigmoid(g)) * u).astype(jnp.bfloat16)
    shared = lax.dot_general(h, sd_s[...], (((1,), (1,)), ((), ())),
                             preferred_element_type=jnp.float32)
    out_ref[...] = shared + ya_ref[...] + yb_ref[...]


def kernel(hidden_states, router_w, router_bias, Wg, Wu, Wd, Sg, Su, Sd):
    orig_shape = hidden_states.shape
    x = hidden_states.reshape(-1, _D)

    topk_idx, topk_w, xc = pl.pallas_call(
        _router_body,
        out_shape=(jax.ShapeDtypeStruct((_T, _K), jnp.int32),
                   jax.ShapeDtypeStruct((_T, _K), jnp.float32),
                   jax.ShapeDtypeStruct((_T, _D), jnp.float32)),
    )(x, router_w, router_bias.reshape(1, _E))

    e_flat = topk_idx.reshape(-1)
    order = jnp.argsort(e_flat, stable=True).astype(jnp.int32)
    e_sorted = e_flat[order]
    off = jnp.searchsorted(
        e_sorted, jnp.arange(_E + 1, dtype=e_sorted.dtype), side='left'
    ).astype(jnp.int32)
    counts = off[1:] - off[:-1]
    psize = ((counts + (_B - 1)) // _B) * _B
    poff = jnp.concatenate(
        [jnp.zeros(1, jnp.int32), jnp.cumsum(psize).astype(jnp.int32)])
    pp = jnp.arange(_N, dtype=jnp.int32) - off[e_sorted] + poff[e_sorted]
    tok_pad = (jnp.arange(_NP, dtype=jnp.int32) % _T).at[pp].set(
        (order // _K).astype(jnp.int32))
    w_pad = jnp.zeros(_NP, jnp.float32).at[pp].set(
        topk_w.reshape(-1)[order])
    inv = jnp.zeros(_N, jnp.int32).at[order].set(pp)
    inv0 = inv[0::2]
    inv1 = inv[1::2]
    block_expert = jnp.clip(
        jnp.searchsorted(poff, jnp.arange(_NBP, dtype=jnp.int32) * _B,
                         side='right').astype(jnp.int32) - 1,
        0, _E - 1)

    xs = _make_sc_gather(_NP, (_NP, _D))(xc, tok_pad)

    ys = pl.pallas_call(
        _gemm_body,
        grid_spec=pltpu.PrefetchScalarGridSpec(
            num_scalar_prefetch=1,
            grid=(_NBP,),
            in_specs=[
                pl.BlockSpec((1, 1, _B), lambda g, be: (g, 0, 0)),
                pl.BlockSpec((_B, _D), lambda g, be: (g, 0)),
                pl.BlockSpec((1, _FF, _D), lambda g, be: (be[g], 0, 0)),
                pl.BlockSpec((1, _FF, _D), lambda g, be: (be[g], 0, 0)),
                pl.BlockSpec((1, _D, _FF), lambda g, be: (be[g], 0, 0)),
            ],
            out_specs=pl.BlockSpec((_B, _D), lambda g, be: (g, 0)),
            scratch_shapes=[
                pltpu.VMEM((_FF, _D), jnp.bfloat16),
                pltpu.VMEM((_FF, _D), jnp.bfloat16),
                pltpu.VMEM((_D, _FF), jnp.bfloat16),
            ],
        ),
        out_shape=jax.ShapeDtypeStruct((_NP, _D), jnp.float32),
        compiler_params=pltpu.CompilerParams(
            dimension_semantics=("arbitrary",),
        ),
    )(block_expert, w_pad.reshape(_NBP, 1, _B), xs, Wg, Wu, Wd)

    inv_cat = jnp.concatenate([inv0, inv1])
    ysab = _make_sc_gather(2 * _T, (2 * _T, _D))(ys, inv_cat)

    tb = _T // 4
    out = pl.pallas_call(
        _shared_body,
        grid=(4,),
        in_specs=[
            pl.BlockSpec((tb, _D), lambda t: (t, 0)),
            pl.BlockSpec((_FF, _D), lambda t: (0, 0)),
            pl.BlockSpec((_FF, _D), lambda t: (0, 0)),
            pl.BlockSpec((_D, _FF), lambda t: (0, 0)),
            pl.BlockSpec((tb, _D), lambda t: (t, 0)),
            pl.BlockSpec((tb, _D), lambda t: (t + _T // tb, 0)),
        ],
        out_specs=pl.BlockSpec((tb, _D), lambda t: (t, 0)),
        out_shape=jax.ShapeDtypeStruct((_T, _D), jnp.float32),
        scratch_shapes=[
            pltpu.VMEM((_FF, _D), jnp.bfloat16),
            pltpu.VMEM((_FF, _D), jnp.bfloat16),
            pltpu.VMEM((_D, _FF), jnp.bfloat16),
        ],
    )(x, Sg, Su, Sd, ysab, ysab)

    return out.reshape(orig_shape)


# speedup vs baseline: 1.3641x; 1.3641x over previous
import functools

import jax
import jax.numpy as jnp
from jax import lax
from jax.experimental import pallas as pl
from jax.experimental.pallas import tpu as pltpu
from jax.experimental.pallas import tpu_sc as plsc

_D = 1024
_DH = _D // 2
_FF = 512
_E = 8
_K = 2
_T = 2048
_N = _T * _K
_B = 128
_NP = _N + _E * _B
_NBP = _NP // _B
_NC = 2
_NS = 16
_NW = _NC * _NS


def _rne16(b):
    return (b + jnp.uint32(0x7FFF) + ((b >> 16) & jnp.uint32(1))) >> 16


def _pack_pairs(a):
    bits = pltpu.bitcast(a, jnp.uint32)
    lo = _rne16(bits[:, :_DH])
    hi = _rne16(bits[:, _DH:])
    return pltpu.bitcast(lo | (hi << 16), jnp.float32)


def _unpack_pairs(pk):
    bits = pltpu.bitcast(pk, jnp.uint32)
    lo = pltpu.bitcast(bits << 16, jnp.float32).astype(jnp.bfloat16)
    hi = pltpu.bitcast(bits & jnp.uint32(0xFFFF0000),
                       jnp.float32).astype(jnp.bfloat16)
    return lo, hi


_DN = (((1,), (1,)), ((), ()))


def _cumsum0(c, t):
    s = 1
    while s < t:
        c = c + jnp.concatenate(
            [jnp.zeros((s, c.shape[1]), c.dtype), c[:-s]], axis=0)
        s *= 2
    return c


def _router_body(x_ref, rw_ref, rb_ref, pp_ref, w_ref, cnt_ref, xpk_ref):
    x = x_ref[...]
    rw = rw_ref[...]
    t = x.shape[0]
    logits = lax.dot_general(x, rw, _DN, preferred_element_type=jnp.float32)
    scores = jax.nn.sigmoid(logits)
    sc = scores + rb_ref[...]
    idx8 = lax.broadcasted_iota(jnp.int32, (t, _E), 1)
    m1 = jnp.max(sc, axis=1, keepdims=True)
    i1 = jnp.min(jnp.where(sc >= m1, idx8, _E), axis=1, keepdims=True)
    w1 = jnp.sum(jnp.where(idx8 == i1, scores, 0.0), axis=1, keepdims=True)
    sc2 = jnp.where(idx8 == i1, -jnp.inf, sc)
    m2 = jnp.max(sc2, axis=1, keepdims=True)
    i2 = jnp.min(jnp.where(sc2 >= m2, idx8, _E), axis=1, keepdims=True)
    w2 = jnp.sum(jnp.where(idx8 == i2, scores, 0.0), axis=1, keepdims=True)
    denom = w1 + w2 + 1e-20
    w_ref[...] = jnp.concatenate([w1 / denom, w2 / denom], axis=1)

    oh1 = idx8 == i1
    oh2 = idx8 == i2
    cnt = oh1.astype(jnp.int32) + oh2.astype(jnp.int32)
    inc = _cumsum0(cnt, t)
    excl = inc - cnt
    counts = inc[t - 1:t, :]
    psize = ((counts + (_B - 1)) // _B) * _B
    poffx = jnp.zeros_like(psize)
    for e in range(1, _E):
        poffx = poffx + jnp.concatenate(
            [jnp.zeros((1, e), jnp.int32), psize[:, :_E - e]], axis=1)
    base = poffx + excl
    pp0 = jnp.sum(jnp.where(oh1, base, 0), axis=1, keepdims=True)
    pp1 = jnp.sum(jnp.where(oh2, base, 0), axis=1, keepdims=True)
    pp_ref[...] = jnp.concatenate([pp0, pp1], axis=1)
    cnt_ref[...] = counts
    xpk_ref[...] = _pack_pairs(x)


def _sc_gather_body(total_rows, cols, x_hbm, tok_hbm, xs_hbm,
                    idx0_v, idx1_v, idx2_v, idx3_v, buf0, buf1, sem0, sem1):
    wid = lax.axis_index("s") * _NC + lax.axis_index("c")
    rows_per_w = total_rows // _NW
    chunk = rows_per_w // 4
    base = wid * rows_per_w
    idx = (idx0_v, idx1_v, idx2_v, idx3_v)
    for j in range(4):
        pltpu.sync_copy(tok_hbm.at[pl.ds(base + j * chunk, chunk)], idx[j])
    cp0 = pltpu.async_copy(x_hbm.at[idx0_v], buf0, sem0)
    cp1 = pltpu.async_copy(x_hbm.at[idx1_v], buf1, sem1)
    cp0.wait()
    pltpu.sync_copy(buf0, xs_hbm.at[pl.ds(base, chunk)])
    cp2 = pltpu.async_copy(x_hbm.at[idx2_v], buf0, sem0)
    cp1.wait()
    pltpu.sync_copy(buf1, xs_hbm.at[pl.ds(base + chunk, chunk)])
    cp3 = pltpu.async_copy(x_hbm.at[idx3_v], buf1, sem1)
    cp2.wait()
    pltpu.sync_copy(buf0, xs_hbm.at[pl.ds(base + 2 * chunk, chunk)])
    cp3.wait()
    pltpu.sync_copy(buf1, xs_hbm.at[pl.ds(base + 3 * chunk, chunk)])


def _make_sc_gather(total_rows, cols):
    chunk = total_rows // _NW // 4
    return functools.partial(
        pl.kernel,
        mesh=plsc.VectorSubcoreMesh(core_axis_name="c", subcore_axis_name="s"),
        out_type=jax.ShapeDtypeStruct((total_rows, cols), jnp.float32),
        scratch_types=[
            pltpu.VMEM((chunk,), jnp.int32),
            pltpu.VMEM((chunk,), jnp.int32),
            pltpu.VMEM((chunk,), jnp.int32),
            pltpu.VMEM((chunk,), jnp.int32),
            pltpu.VMEM((chunk, cols), jnp.float32),
            pltpu.VMEM((chunk, cols), jnp.float32),
            pltpu.SemaphoreType.DMA,
            pltpu.SemaphoreType.DMA,
        ],
    )(functools.partial(_sc_gather_body, total_rows, cols))


def _gemm_body(be_ref, wsc_ref, xs_ref, wg_ref, wu_ref, wd_ref, ys_ref,
               wg_s, wu_s, wd_s):
    g = pl.program_id(0)
    prev = jnp.where(g == 0, -1, be_ref[jnp.maximum(g - 1, 0)])

    @pl.when(be_ref[g] != prev)
    def _():
        wg_s[...] = wg_ref[0].astype(jnp.bfloat16)
        wu_s[...] = wu_ref[0].astype(jnp.bfloat16)
        wd_s[...] = wd_ref[0].astype(jnp.bfloat16)

    xlo, xhi = _unpack_pairs(xs_ref[...])
    gg = (lax.dot_general(xlo, wg_s[:, :_DH], _DN,
                          preferred_element_type=jnp.float32)
          + lax.dot_general(xhi, wg_s[:, _DH:], _DN,
                            preferred_element_type=jnp.float32))
    uu = (lax.dot_general(xlo, wu_s[:, :_DH], _DN,
                          preferred_element_type=jnp.float32)
          + lax.dot_general(xhi, wu_s[:, _DH:], _DN,
                            preferred_element_type=jnp.float32))
    h = ((gg * jax.nn.sigmoid(gg)) * uu
         * wsc_ref[0, 0, :][:, None]).astype(jnp.bfloat16)
    y = lax.dot_general(h, wd_s[...], _DN,
                        preferred_element_type=jnp.float32)
    ys_ref[...] = _pack_pairs(y)


def _shared_body(x_ref, sg_ref, su_ref, sd_ref, ya_ref, yb_ref, out_ref,
                 sg_s, su_s, sd_s):
    @pl.when(pl.program_id(0) == 0)
    def _():
        sg_s[...] = sg_ref[...].astype(jnp.bfloat16)
        su_s[...] = su_ref[...].astype(jnp.bfloat16)
        sd_s[...] = sd_ref[...].astype(jnp.bfloat16)

    xb = x_ref[...].astype(jnp.bfloat16)
    g = lax.dot_general(xb, sg_s[...], _DN,
                        preferred_element_type=jnp.float32)
    u = lax.dot_general(xb, su_s[...], _DN,
                        preferred_element_type=jnp.float32)
    h = ((g * jax.nn.sigmoid(g)) * u).astype(jnp.bfloat16)
    shared = lax.dot_general(h, sd_s[...], _DN,
                             preferred_element_type=jnp.float32)
    alo, ahi = _unpack_pairs(ya_ref[...])
    blo, bhi = _unpack_pairs(yb_ref[...])
    out_ref[:, :_DH] = (shared[:, :_DH] + alo.astype(jnp.float32)
                        + blo.astype(jnp.float32))
    out_ref[:, _DH:] = (shared[:, _DH:] + ahi.astype(jnp.float32)
                        + bhi.astype(jnp.float32))


def kernel(hidden_states, router_w, router_bias, Wg, Wu, Wd, Sg, Su, Sd):
    orig_shape = hidden_states.shape
    x = hidden_states.reshape(-1, _D)

    pp, topk_w, cnt, xpk = pl.pallas_call(
        _router_body,
        out_shape=(jax.ShapeDtypeStruct((_T, _K), jnp.int32),
                   jax.ShapeDtypeStruct((_T, _K), jnp.float32),
                   jax.ShapeDtypeStruct((1, _E), jnp.int32),
                   jax.ShapeDtypeStruct((_T, _DH), jnp.float32)),
    )(x, router_w, router_bias.reshape(1, _E))

    counts = cnt.reshape(_E)
    psize = ((counts + (_B - 1)) // _B) * _B
    poff = jnp.concatenate(
        [jnp.zeros(1, jnp.int32), jnp.cumsum(psize).astype(jnp.int32)])
    pp_flat = pp.reshape(-1)
    tok_pad = (jnp.arange(_NP, dtype=jnp.int32) % _T).at[pp_flat].set(
        jnp.arange(_N, dtype=jnp.int32) // _K)
    w_pad = jnp.zeros(_NP, jnp.float32).at[pp_flat].set(topk_w.reshape(-1))
    block_expert = jnp.clip(
        jnp.searchsorted(poff, jnp.arange(_NBP, dtype=jnp.int32) * _B,
                         side='right').astype(jnp.int32) - 1,
        0, _E - 1)
    inv_cat = jnp.concatenate([pp[:, 0], pp[:, 1]])

    xs = _make_sc_gather(_NP, _DH)(xpk, tok_pad)

    ys = pl.pallas_call(
        _gemm_body,
        grid_spec=pltpu.PrefetchScalarGridSpec(
            num_scalar_prefetch=1,
            grid=(_NBP,),
            in_specs=[
                pl.BlockSpec((1, 1, _B), lambda g, be: (g, 0, 0)),
                pl.BlockSpec((_B, _DH), lambda g, be: (g, 0)),
                pl.BlockSpec((1, _FF, _D), lambda g, be: (be[g], 0, 0)),
                pl.BlockSpec((1, _FF, _D), lambda g, be: (be[g], 0, 0)),
                pl.BlockSpec((1, _D, _FF), lambda g, be: (be[g], 0, 0)),
            ],
            out_specs=pl.BlockSpec((_B, _DH), lambda g, be: (g, 0)),
            scratch_shapes=[
                pltpu.VMEM((_FF, _D), jnp.bfloat16),
                pltpu.VMEM((_FF, _D), jnp.bfloat16),
                pltpu.VMEM((_D, _FF), jnp.bfloat16),
            ],
        ),
        out_shape=jax.ShapeDtypeStruct((_NP, _DH), jnp.float32),
        compiler_params=pltpu.CompilerParams(
            dimension_semantics=("arbitrary",),
        ),
    )(block_expert, w_pad.reshape(_NBP, 1, _B), xs, Wg, Wu, Wd)

    ysab = _make_sc_gather(2 * _T, _DH)(ys, inv_cat)

    tb = _T // 4
    out = pl.pallas_call(
        _shared_body,
        grid=(4,),
        in_specs=[
            pl.BlockSpec((tb, _D), lambda t: (t, 0)),
            pl.BlockSpec((_FF, _D), lambda t: (0, 0)),
            pl.BlockSpec((_FF, _D), lambda t: (0, 0)),
            pl.BlockSpec((_D, _FF), lambda t: (0, 0)),
            pl.BlockSpec((tb, _DH), lambda t: (t, 0)),
            pl.BlockSpec((tb, _DH), lambda t: (t + _T // tb, 0)),
        ],
        out_specs=pl.BlockSpec((tb, _D), lambda t: (t, 0)),
        out_shape=jax.ShapeDtypeStruct((_T, _D), jnp.float32),
        scratch_shapes=[
            pltpu.VMEM((_FF, _D), jnp.bfloat16),
            pltpu.VMEM((_FF, _D), jnp.bfloat16),
            pltpu.VMEM((_D, _FF), jnp.bfloat16),
        ],
    )(x, Sg, Su, Sd, ysab, ysab)

    return out.reshape(orig_shape)


# speedup vs baseline: 1.5709x; 1.1516x over previous
import functools

import jax
import jax.numpy as jnp
from jax import lax
from jax.experimental import pallas as pl
from jax.experimental.pallas import tpu as pltpu
from jax.experimental.pallas import tpu_sc as plsc

_D = 1024
_DH = _D // 2
_FF = 512
_E = 8
_K = 2
_T = 2048
_N = _T * _K
_B = 256
_NP = _N + _E * _B
_NBP = _NP // _B
_NC = 2
_NS = 16
_NW = _NC * _NS


def _rne16(b):
    return (b + jnp.uint32(0x7FFF) + ((b >> 16) & jnp.uint32(1))) >> 16


def _pack_pairs(a):
    bits = pltpu.bitcast(a, jnp.uint32)
    lo = _rne16(bits[:, :_DH])
    hi = _rne16(bits[:, _DH:])
    return pltpu.bitcast(lo | (hi << 16), jnp.float32)


def _unpack_pairs(pk):
    bits = pltpu.bitcast(pk, jnp.uint32)
    lo = pltpu.bitcast(bits << 16, jnp.float32).astype(jnp.bfloat16)
    hi = pltpu.bitcast(bits & jnp.uint32(0xFFFF0000),
                       jnp.float32).astype(jnp.bfloat16)
    return lo, hi


_DN = (((1,), (1,)), ((), ()))


def _cumsum0(c, t):
    s = 1
    while s < t:
        c = c + jnp.concatenate(
            [jnp.zeros((s, c.shape[1]), c.dtype), c[:-s]], axis=0)
        s *= 2
    return c


def _router_body(x_ref, rw_ref, rb_ref, pp_ref, w_ref, cnt_ref, xpk_ref):
    x = x_ref[...]
    rw = rw_ref[...]
    t = x.shape[0]
    logits = lax.dot_general(x, rw, _DN, preferred_element_type=jnp.float32)
    scores = jax.nn.sigmoid(logits)
    sc = scores + rb_ref[...]
    idx8 = lax.broadcasted_iota(jnp.int32, (t, _E), 1)
    m1 = jnp.max(sc, axis=1, keepdims=True)
    i1 = jnp.min(jnp.where(sc >= m1, idx8, _E), axis=1, keepdims=True)
    w1 = jnp.sum(jnp.where(idx8 == i1, scores, 0.0), axis=1, keepdims=True)
    sc2 = jnp.where(idx8 == i1, -jnp.inf, sc)
    m2 = jnp.max(sc2, axis=1, keepdims=True)
    i2 = jnp.min(jnp.where(sc2 >= m2, idx8, _E), axis=1, keepdims=True)
    w2 = jnp.sum(jnp.where(idx8 == i2, scores, 0.0), axis=1, keepdims=True)
    denom = w1 + w2 + 1e-20
    w_ref[...] = jnp.concatenate([w1 / denom, w2 / denom], axis=1)

    oh1 = idx8 == i1
    oh2 = idx8 == i2
    cnt = oh1.astype(jnp.int32) + oh2.astype(jnp.int32)
    inc = _cumsum0(cnt, t)
    excl = inc - cnt
    counts = inc[t - 1:t, :]
    psize = ((counts + (_B - 1)) // _B) * _B
    poffx = jnp.zeros_like(psize)
    for e in range(1, _E):
        poffx = poffx + jnp.concatenate(
            [jnp.zeros((1, e), jnp.int32), psize[:, :_E - e]], axis=1)
    base = poffx + excl
    pp0 = jnp.sum(jnp.where(oh1, base, 0), axis=1, keepdims=True)
    pp1 = jnp.sum(jnp.where(oh2, base, 0), axis=1, keepdims=True)
    pp_ref[...] = jnp.concatenate([pp0, pp1], axis=1)
    cnt_ref[...] = counts
    xpk_ref[...] = _pack_pairs(x)


def _sc_gather_body(total_rows, cols, x_hbm, tok_hbm, xs_hbm,
                    idx0_v, idx1_v, idx2_v, idx3_v, buf0, buf1, sem0, sem1):
    wid = lax.axis_index("s") * _NC + lax.axis_index("c")
    rows_per_w = total_rows // _NW
    chunk = rows_per_w // 4
    base = wid * rows_per_w
    idx = (idx0_v, idx1_v, idx2_v, idx3_v)
    for j in range(4):
        pltpu.sync_copy(tok_hbm.at[pl.ds(base + j * chunk, chunk)], idx[j])
    cp0 = pltpu.async_copy(x_hbm.at[idx0_v], buf0, sem0)
    cp1 = pltpu.async_copy(x_hbm.at[idx1_v], buf1, sem1)
    cp0.wait()
    pltpu.sync_copy(buf0, xs_hbm.at[pl.ds(base, chunk)])
    cp2 = pltpu.async_copy(x_hbm.at[idx2_v], buf0, sem0)
    cp1.wait()
    pltpu.sync_copy(buf1, xs_hbm.at[pl.ds(base + chunk, chunk)])
    cp3 = pltpu.async_copy(x_hbm.at[idx3_v], buf1, sem1)
    cp2.wait()
    pltpu.sync_copy(buf0, xs_hbm.at[pl.ds(base + 2 * chunk, chunk)])
    cp3.wait()
    pltpu.sync_copy(buf1, xs_hbm.at[pl.ds(base + 3 * chunk, chunk)])


def _make_sc_gather(total_rows, cols):
    chunk = total_rows // _NW // 4
    return functools.partial(
        pl.kernel,
        mesh=plsc.VectorSubcoreMesh(core_axis_name="c", subcore_axis_name="s"),
        out_type=jax.ShapeDtypeStruct((total_rows, cols), jnp.float32),
        scratch_types=[
            pltpu.VMEM((chunk,), jnp.int32),
            pltpu.VMEM((chunk,), jnp.int32),
            pltpu.VMEM((chunk,), jnp.int32),
            pltpu.VMEM((chunk,), jnp.int32),
            pltpu.VMEM((chunk, cols), jnp.float32),
            pltpu.VMEM((chunk, cols), jnp.float32),
            pltpu.SemaphoreType.DMA,
            pltpu.SemaphoreType.DMA,
        ],
    )(functools.partial(_sc_gather_body, total_rows, cols))


def _gemm_body(be_ref, wsc_ref, xs_ref, wg_ref, wu_ref, wd_ref, ys_ref,
               wg_s, wu_s, wd_s):
    g = pl.program_id(0)
    prev = jnp.where(g == 0, -1, be_ref[jnp.maximum(g - 1, 0)])

    @pl.when(be_ref[g] != prev)
    def _():
        wg_s[...] = wg_ref[0].astype(jnp.bfloat16)
        wu_s[...] = wu_ref[0].astype(jnp.bfloat16)
        wd_s[...] = wd_ref[0].astype(jnp.bfloat16)

    xlo, xhi = _unpack_pairs(xs_ref[...])
    gg = (lax.dot_general(xlo, wg_s[:, :_DH], _DN,
                          preferred_element_type=jnp.float32)
          + lax.dot_general(xhi, wg_s[:, _DH:], _DN,
                            preferred_element_type=jnp.float32))
    uu = (lax.dot_general(xlo, wu_s[:, :_DH], _DN,
                          preferred_element_type=jnp.float32)
          + lax.dot_general(xhi, wu_s[:, _DH:], _DN,
                            preferred_element_type=jnp.float32))
    h = ((gg * jax.nn.sigmoid(gg)) * uu
         * wsc_ref[0, 0, :][:, None]).astype(jnp.bfloat16)
    y = lax.dot_general(h, wd_s[...], _DN,
                        preferred_element_type=jnp.float32)
    ys_ref[...] = _pack_pairs(y)


def _shared_body(x_ref, sg_ref, su_ref, sd_ref, ya_ref, yb_ref, out_ref,
                 sg_s, su_s, sd_s):
    @pl.when(pl.program_id(0) == 0)
    def _():
        sg_s[...] = sg_ref[...].astype(jnp.bfloat16)
        su_s[...] = su_ref[...].astype(jnp.bfloat16)
        sd_s[...] = sd_ref[...].astype(jnp.bfloat16)

    xb = x_ref[...].astype(jnp.bfloat16)
    g = lax.dot_general(xb, sg_s[...], _DN,
                        preferred_element_type=jnp.float32)
    u = lax.dot_general(xb, su_s[...], _DN,
                        preferred_element_type=jnp.float32)
    h = ((g * jax.nn.sigmoid(g)) * u).astype(jnp.bfloat16)
    shared = lax.dot_general(h, sd_s[...], _DN,
                             preferred_element_type=jnp.float32)
    alo, ahi = _unpack_pairs(ya_ref[...])
    blo, bhi = _unpack_pairs(yb_ref[...])
    out_ref[:, :_DH] = (shared[:, :_DH] + alo.astype(jnp.float32)
                        + blo.astype(jnp.float32))
    out_ref[:, _DH:] = (shared[:, _DH:] + ahi.astype(jnp.float32)
                        + bhi.astype(jnp.float32))


def kernel(hidden_states, router_w, router_bias, Wg, Wu, Wd, Sg, Su, Sd):
    orig_shape = hidden_states.shape
    x = hidden_states.reshape(-1, _D)

    pp, topk_w, cnt, xpk = pl.pallas_call(
        _router_body,
        out_shape=(jax.ShapeDtypeStruct((_T, _K), jnp.int32),
                   jax.ShapeDtypeStruct((_T, _K), jnp.float32),
                   jax.ShapeDtypeStruct((1, _E), jnp.int32),
                   jax.ShapeDtypeStruct((_T, _DH), jnp.float32)),
    )(x, router_w, router_bias.reshape(1, _E))

    counts = cnt.reshape(_E)
    psize = ((counts + (_B - 1)) // _B) * _B
    poff = jnp.concatenate(
        [jnp.zeros(1, jnp.int32), jnp.cumsum(psize).astype(jnp.int32)])
    pp_flat = pp.reshape(-1)
    tok_pad = (jnp.arange(_NP, dtype=jnp.int32) % _T).at[pp_flat].set(
        jnp.arange(_N, dtype=jnp.int32) // _K)
    w_pad = jnp.zeros(_NP, jnp.float32).at[pp_flat].set(topk_w.reshape(-1))
    block_expert = jnp.clip(
        jnp.searchsorted(poff, jnp.arange(_NBP, dtype=jnp.int32) * _B,
                         side='right').astype(jnp.int32) - 1,
        0, _E - 1)
    inv_cat = jnp.concatenate([pp[:, 0], pp[:, 1]])

    xs = _make_sc_gather(_NP, _DH)(xpk, tok_pad)

    ys = pl.pallas_call(
        _gemm_body,
        grid_spec=pltpu.PrefetchScalarGridSpec(
            num_scalar_prefetch=1,
            grid=(_NBP,),
            in_specs=[
                pl.BlockSpec((1, 1, _B), lambda g, be: (g, 0, 0)),
                pl.BlockSpec((_B, _DH), lambda g, be: (g, 0)),
                pl.BlockSpec((1, _FF, _D), lambda g, be: (be[g], 0, 0)),
                pl.BlockSpec((1, _FF, _D), lambda g, be: (be[g], 0, 0)),
                pl.BlockSpec((1, _D, _FF), lambda g, be: (be[g], 0, 0)),
            ],
            out_specs=pl.BlockSpec((_B, _DH), lambda g, be: (g, 0)),
            scratch_shapes=[
                pltpu.VMEM((_FF, _D), jnp.bfloat16),
                pltpu.VMEM((_FF, _D), jnp.bfloat16),
                pltpu.VMEM((_D, _FF), jnp.bfloat16),
            ],
        ),
        out_shape=jax.ShapeDtypeStruct((_NP, _DH), jnp.float32),
        compiler_params=pltpu.CompilerParams(
            dimension_semantics=("arbitrary",),
        ),
    )(block_expert, w_pad.reshape(_NBP, 1, _B), xs, Wg, Wu, Wd)

    ysab = _make_sc_gather(2 * _T, _DH)(ys, inv_cat)

    tb = _T // 4
    out = pl.pallas_call(
        _shared_body,
        grid=(4,),
        in_specs=[
            pl.BlockSpec((tb, _D), lambda t: (t, 0)),
            pl.BlockSpec((_FF, _D), lambda t: (0, 0)),
            pl.BlockSpec((_FF, _D), lambda t: (0, 0)),
            pl.BlockSpec((_D, _FF), lambda t: (0, 0)),
            pl.BlockSpec((tb, _DH), lambda t: (t, 0)),
            pl.BlockSpec((tb, _DH), lambda t: (t + _T // tb, 0)),
        ],
        out_specs=pl.BlockSpec((tb, _D), lambda t: (t, 0)),
        out_shape=jax.ShapeDtypeStruct((_T, _D), jnp.float32),
        scratch_shapes=[
            pltpu.VMEM((_FF, _D), jnp.bfloat16),
            pltpu.VMEM((_FF, _D), jnp.bfloat16),
            pltpu.VMEM((_D, _FF), jnp.bfloat16),
        ],
    )(x, Sg, Su, Sd, ysab, ysab)

    return out.reshape(orig_shape)


# speedup vs baseline: 1.5772x; 1.0040x over previous
import functools

import jax
import jax.numpy as jnp
from jax import lax
from jax.experimental import pallas as pl
from jax.experimental.pallas import tpu as pltpu
from jax.experimental.pallas import tpu_sc as plsc

_D = 1024
_DH = _D // 2
_FF = 512
_E = 8
_K = 2
_T = 2048
_N = _T * _K
_B = 256
_NP = _N + _E * _B
_NBP = _NP // _B
_NC = 2
_NS = 16
_NW = _NC * _NS


def _rne16(b):
    return (b + jnp.uint32(0x7FFF) + ((b >> 16) & jnp.uint32(1))) >> 16


def _pack_pairs(a):
    bits = pltpu.bitcast(a, jnp.uint32)
    lo = _rne16(bits[:, :_DH])
    hi = _rne16(bits[:, _DH:])
    return pltpu.bitcast(lo | (hi << 16), jnp.float32)


def _unpack_pairs(pk):
    bits = pltpu.bitcast(pk, jnp.uint32)
    lo = pltpu.bitcast(bits << 16, jnp.float32).astype(jnp.bfloat16)
    hi = pltpu.bitcast(bits & jnp.uint32(0xFFFF0000),
                       jnp.float32).astype(jnp.bfloat16)
    return lo, hi


_DN = (((1,), (1,)), ((), ()))


def _cumsum0(c, t):
    s = 1
    while s < t:
        c = c + jnp.concatenate(
            [jnp.zeros((s, c.shape[1]), c.dtype), c[:-s]], axis=0)
        s *= 2
    return c


def _router_body(x_ref, rw_ref, rb_ref, pp_ref, w_ref, cnt_ref, xpk_ref):
    x = x_ref[...]
    rw = rw_ref[...]
    t = x.shape[0]
    logits = lax.dot_general(x, rw, _DN, preferred_element_type=jnp.float32)
    scores = jax.nn.sigmoid(logits)
    sc = scores + rb_ref[...]
    idx8 = lax.broadcasted_iota(jnp.int32, (t, _E), 1)
    m1 = jnp.max(sc, axis=1, keepdims=True)
    i1 = jnp.min(jnp.where(sc >= m1, idx8, _E), axis=1, keepdims=True)
    w1 = jnp.sum(jnp.where(idx8 == i1, scores, 0.0), axis=1, keepdims=True)
    sc2 = jnp.where(idx8 == i1, -jnp.inf, sc)
    m2 = jnp.max(sc2, axis=1, keepdims=True)
    i2 = jnp.min(jnp.where(sc2 >= m2, idx8, _E), axis=1, keepdims=True)
    w2 = jnp.sum(jnp.where(idx8 == i2, scores, 0.0), axis=1, keepdims=True)
    denom = w1 + w2 + 1e-20
    w_ref[...] = jnp.concatenate([w1 / denom, w2 / denom], axis=1)

    oh1 = idx8 == i1
    oh2 = idx8 == i2
    cnt = oh1.astype(jnp.int32) + oh2.astype(jnp.int32)
    inc = _cumsum0(cnt, t)
    excl = inc - cnt
    counts = inc[t - 1:t, :]
    psize = ((counts + (_B - 1)) // _B) * _B
    poffx = jnp.zeros_like(psize)
    for e in range(1, _E):
        poffx = poffx + jnp.concatenate(
            [jnp.zeros((1, e), jnp.int32), psize[:, :_E - e]], axis=1)
    base = poffx + excl
    pp0 = jnp.sum(jnp.where(oh1, base, 0), axis=1, keepdims=True)
    pp1 = jnp.sum(jnp.where(oh2, base, 0), axis=1, keepdims=True)
    pp_ref[...] = jnp.concatenate([pp0, pp1], axis=1)
    cnt_ref[...] = counts
    xpk_ref[...] = _pack_pairs(x)


def _sc_gather_body(total_rows, cols, x_hbm, tok_hbm, xs_hbm,
                    idx0_v, idx1_v, idx2_v, idx3_v, buf0, buf1, sem0, sem1):
    wid = lax.axis_index("s") * _NC + lax.axis_index("c")
    rows_per_w = total_rows // _NW
    chunk = rows_per_w // 4
    base = wid * rows_per_w
    idx = (idx0_v, idx1_v, idx2_v, idx3_v)
    for j in range(4):
        pltpu.sync_copy(tok_hbm.at[pl.ds(base + j * chunk, chunk)], idx[j])
    cp0 = pltpu.async_copy(x_hbm.at[idx0_v], buf0, sem0)
    cp1 = pltpu.async_copy(x_hbm.at[idx1_v], buf1, sem1)
    cp0.wait()
    pltpu.sync_copy(buf0, xs_hbm.at[pl.ds(base, chunk)])
    cp2 = pltpu.async_copy(x_hbm.at[idx2_v], buf0, sem0)
    cp1.wait()
    pltpu.sync_copy(buf1, xs_hbm.at[pl.ds(base + chunk, chunk)])
    cp3 = pltpu.async_copy(x_hbm.at[idx3_v], buf1, sem1)
    cp2.wait()
    pltpu.sync_copy(buf0, xs_hbm.at[pl.ds(base + 2 * chunk, chunk)])
    cp3.wait()
    pltpu.sync_copy(buf1, xs_hbm.at[pl.ds(base + 3 * chunk, chunk)])


def _make_sc_gather(total_rows, cols):
    chunk = total_rows // _NW // 4
    return functools.partial(
        pl.kernel,
        mesh=plsc.VectorSubcoreMesh(core_axis_name="c", subcore_axis_name="s"),
        out_type=jax.ShapeDtypeStruct((total_rows, cols), jnp.float32),
        scratch_types=[
            pltpu.VMEM((chunk,), jnp.int32),
            pltpu.VMEM((chunk,), jnp.int32),
            pltpu.VMEM((chunk,), jnp.int32),
            pltpu.VMEM((chunk,), jnp.int32),
            pltpu.VMEM((chunk, cols), jnp.float32),
            pltpu.VMEM((chunk, cols), jnp.float32),
            pltpu.SemaphoreType.DMA,
            pltpu.SemaphoreType.DMA,
        ],
    )(functools.partial(_sc_gather_body, total_rows, cols))


def _gemm_body(be_ref, wsc_ref, xs_ref, wg_ref, wu_ref, wd_ref, ys_ref,
               wg_s, wu_s, wd_s):
    g = pl.program_id(0)
    prev = jnp.where(g == 0, -1, be_ref[jnp.maximum(g - 1, 0)])

    @pl.when(be_ref[g] != prev)
    def _():
        wg_s[...] = wg_ref[0].astype(jnp.bfloat16)
        wu_s[...] = wu_ref[0].astype(jnp.bfloat16)
        wd_s[...] = wd_ref[0].astype(jnp.bfloat16)

    xlo, xhi = _unpack_pairs(xs_ref[...])
    gg = (lax.dot_general(xlo, wg_s[:, :_DH], _DN,
                          preferred_element_type=jnp.float32)
          + lax.dot_general(xhi, wg_s[:, _DH:], _DN,
                            preferred_element_type=jnp.float32))
    uu = (lax.dot_general(xlo, wu_s[:, :_DH], _DN,
                          preferred_element_type=jnp.float32)
          + lax.dot_general(xhi, wu_s[:, _DH:], _DN,
                            preferred_element_type=jnp.float32))
    h = ((gg * jax.nn.sigmoid(gg)) * uu
         * wsc_ref[0, 0, :][:, None]).astype(jnp.bfloat16)
    y = lax.dot_general(h, wd_s[...], _DN,
                        preferred_element_type=jnp.float32)
    ys_ref[...] = _pack_pairs(y)


def _shared_body(x_ref, sg_ref, su_ref, sd_ref, ya_ref, yb_ref, out_ref,
                 sg_s, su_s, sd_s):
    @pl.when(pl.program_id(0) == 0)
    def _():
        sg_s[...] = sg_ref[...].astype(jnp.bfloat16)
        su_s[...] = su_ref[...].astype(jnp.bfloat16)
        sd_s[...] = sd_ref[...].astype(jnp.bfloat16)

    xlo, xhi = _unpack_pairs(x_ref[...])
    g = (lax.dot_general(xlo, sg_s[:, :_DH], _DN,
                         preferred_element_type=jnp.float32)
         + lax.dot_general(xhi, sg_s[:, _DH:], _DN,
                           preferred_element_type=jnp.float32))
    u = (lax.dot_general(xlo, su_s[:, :_DH], _DN,
                         preferred_element_type=jnp.float32)
         + lax.dot_general(xhi, su_s[:, _DH:], _DN,
                           preferred_element_type=jnp.float32))
    h = ((g * jax.nn.sigmoid(g)) * u).astype(jnp.bfloat16)
    shared = lax.dot_general(h, sd_s[...], _DN,
                             preferred_element_type=jnp.float32)
    alo, ahi = _unpack_pairs(ya_ref[...])
    blo, bhi = _unpack_pairs(yb_ref[...])
    out_ref[:, :_DH] = (shared[:, :_DH] + alo.astype(jnp.float32)
                        + blo.astype(jnp.float32))
    out_ref[:, _DH:] = (shared[:, _DH:] + ahi.astype(jnp.float32)
                        + bhi.astype(jnp.float32))


def kernel(hidden_states, router_w, router_bias, Wg, Wu, Wd, Sg, Su, Sd):
    orig_shape = hidden_states.shape
    x = hidden_states.reshape(-1, _D)

    pp, topk_w, cnt, xpk = pl.pallas_call(
        _router_body,
        out_shape=(jax.ShapeDtypeStruct((_T, _K), jnp.int32),
                   jax.ShapeDtypeStruct((_T, _K), jnp.float32),
                   jax.ShapeDtypeStruct((1, _E), jnp.int32),
                   jax.ShapeDtypeStruct((_T, _DH), jnp.float32)),
    )(x, router_w, router_bias.reshape(1, _E))

    counts = cnt.reshape(_E)
    psize = ((counts + (_B - 1)) // _B) * _B
    poff = jnp.concatenate(
        [jnp.zeros(1, jnp.int32), jnp.cumsum(psize).astype(jnp.int32)])
    pp_flat = pp.reshape(-1)
    tok_pad = (jnp.arange(_NP, dtype=jnp.int32) % _T).at[pp_flat].set(
        jnp.arange(_N, dtype=jnp.int32) // _K)
    w_pad = jnp.zeros(_NP, jnp.float32).at[pp_flat].set(topk_w.reshape(-1))
    block_expert = jnp.clip(
        jnp.searchsorted(poff, jnp.arange(_NBP, dtype=jnp.int32) * _B,
                         side='right').astype(jnp.int32) - 1,
        0, _E - 1)
    inv_cat = jnp.concatenate([pp[:, 0], pp[:, 1]])

    xs = _make_sc_gather(_NP, _DH)(xpk, tok_pad)

    ys = pl.pallas_call(
        _gemm_body,
        grid_spec=pltpu.PrefetchScalarGridSpec(
            num_scalar_prefetch=1,
            grid=(_NBP,),
            in_specs=[
                pl.BlockSpec((1, 1, _B), lambda g, be: (g, 0, 0)),
                pl.BlockSpec((_B, _DH), lambda g, be: (g, 0)),
                pl.BlockSpec((1, _FF, _D), lambda g, be: (be[g], 0, 0)),
                pl.BlockSpec((1, _FF, _D), lambda g, be: (be[g], 0, 0)),
                pl.BlockSpec((1, _D, _FF), lambda g, be: (be[g], 0, 0)),
            ],
            out_specs=pl.BlockSpec((_B, _DH), lambda g, be: (g, 0)),
            scratch_shapes=[
                pltpu.VMEM((_FF, _D), jnp.bfloat16),
                pltpu.VMEM((_FF, _D), jnp.bfloat16),
                pltpu.VMEM((_D, _FF), jnp.bfloat16),
            ],
        ),
        out_shape=jax.ShapeDtypeStruct((_NP, _DH), jnp.float32),
        compiler_params=pltpu.CompilerParams(
            dimension_semantics=("arbitrary",),
        ),
    )(block_expert, w_pad.reshape(_NBP, 1, _B), xs, Wg, Wu, Wd)

    ysab = _make_sc_gather(2 * _T, _DH)(ys, inv_cat)

    tb = _T // 4
    out = pl.pallas_call(
        _shared_body,
        grid=(4,),
        in_specs=[
            pl.BlockSpec((tb, _DH), lambda t: (t, 0)),
            pl.BlockSpec((_FF, _D), lambda t: (0, 0)),
            pl.BlockSpec((_FF, _D), lambda t: (0, 0)),
            pl.BlockSpec((_D, _FF), lambda t: (0, 0)),
            pl.BlockSpec((tb, _DH), lambda t: (t, 0)),
            pl.BlockSpec((tb, _DH), lambda t: (t + _T // tb, 0)),
        ],
        out_specs=pl.BlockSpec((tb, _D), lambda t: (t, 0)),
        out_shape=jax.ShapeDtypeStruct((_T, _D), jnp.float32),
        scratch_shapes=[
            pltpu.VMEM((_FF, _D), jnp.bfloat16),
            pltpu.VMEM((_FF, _D), jnp.bfloat16),
            pltpu.VMEM((_D, _FF), jnp.bfloat16),
        ],
    )(xpk, Sg, Su, Sd, ysab, ysab)

    return out.reshape(orig_shape)


# speedup vs baseline: 1.6223x; 1.0286x over previous
import functools

import jax
import jax.numpy as jnp
from jax import lax
from jax.experimental import pallas as pl
from jax.experimental.pallas import tpu as pltpu
from jax.experimental.pallas import tpu_sc as plsc

_D = 1024
_DH = _D // 2
_FF = 512
_E = 8
_K = 2
_T = 2048
_N = _T * _K
_B = 256
_NP = _N + _E * _B
_NBP = _NP // _B
_NC = 2
_NS = 16
_NW = _NC * _NS


def _rne16(b):
    return (b + jnp.uint32(0x7FFF) + ((b >> 16) & jnp.uint32(1))) >> 16


def _pack_pairs(a):
    bits = pltpu.bitcast(a, jnp.uint32)
    lo = _rne16(bits[:, :_DH])
    hi = _rne16(bits[:, _DH:])
    return pltpu.bitcast(lo | (hi << 16), jnp.float32)


def _unpack_pairs(pk):
    bits = pltpu.bitcast(pk, jnp.uint32)
    lo = pltpu.bitcast(bits << 16, jnp.float32).astype(jnp.bfloat16)
    hi = pltpu.bitcast(bits & jnp.uint32(0xFFFF0000),
                       jnp.float32).astype(jnp.bfloat16)
    return lo, hi


_DN = (((1,), (1,)), ((), ()))


def _cumsum0(c, t):
    s = 1
    while s < t:
        c = c + jnp.concatenate(
            [jnp.zeros((s, c.shape[1]), c.dtype), c[:-s]], axis=0)
        s *= 2
    return c


def _router_body(x_ref, rw_ref, rb_ref, pp_ref, w_ref, cnt_ref, xpk_ref):
    x = x_ref[...]
    rw = rw_ref[...]
    t = x.shape[0]
    logits = lax.dot_general(x, rw, _DN, preferred_element_type=jnp.float32)
    scores = jax.nn.sigmoid(logits)
    sc = scores + rb_ref[...]
    idx8 = lax.broadcasted_iota(jnp.int32, (t, _E), 1)
    m1 = jnp.max(sc, axis=1, keepdims=True)
    i1 = jnp.min(jnp.where(sc >= m1, idx8, _E), axis=1, keepdims=True)
    w1 = jnp.sum(jnp.where(idx8 == i1, scores, 0.0), axis=1, keepdims=True)
    sc2 = jnp.where(idx8 == i1, -jnp.inf, sc)
    m2 = jnp.max(sc2, axis=1, keepdims=True)
    i2 = jnp.min(jnp.where(sc2 >= m2, idx8, _E), axis=1, keepdims=True)
    w2 = jnp.sum(jnp.where(idx8 == i2, scores, 0.0), axis=1, keepdims=True)
    denom = w1 + w2 + 1e-20
    w_ref[...] = jnp.concatenate([w1 / denom, w2 / denom], axis=1)

    oh1 = idx8 == i1
    oh2 = idx8 == i2
    cnt = oh1.astype(jnp.int32) + oh2.astype(jnp.int32)
    inc = _cumsum0(cnt, t)
    excl = inc - cnt
    counts = inc[t - 1:t, :]
    psize = ((counts + (_B - 1)) // _B) * _B
    poffx = jnp.zeros_like(psize)
    for e in range(1, _E):
        poffx = poffx + jnp.concatenate(
            [jnp.zeros((1, e), jnp.int32), psize[:, :_E - e]], axis=1)
    base = poffx + excl
    pp0 = jnp.sum(jnp.where(oh1, base, 0), axis=1, keepdims=True)
    pp1 = jnp.sum(jnp.where(oh2, base, 0), axis=1, keepdims=True)
    pp_ref[...] = jnp.concatenate([pp0, pp1], axis=1)
    cnt_ref[...] = counts
    xpk_ref[...] = _pack_pairs(x)


def _sc_gather_body(total_rows, cols, x_hbm, tok_hbm, xs_hbm,
                    idx0_v, idx1_v, idx2_v, idx3_v, buf0, buf1, sem0, sem1):
    wid = lax.axis_index("s") * _NC + lax.axis_index("c")
    rows_per_w = total_rows // _NW
    chunk = rows_per_w // 4
    base = wid * rows_per_w
    idx = (idx0_v, idx1_v, idx2_v, idx3_v)
    for j in range(4):
        pltpu.sync_copy(tok_hbm.at[pl.ds(base + j * chunk, chunk)], idx[j])
    cp0 = pltpu.async_copy(x_hbm.at[idx0_v], buf0, sem0)
    cp1 = pltpu.async_copy(x_hbm.at[idx1_v], buf1, sem1)
    cp0.wait()
    pltpu.sync_copy(buf0, xs_hbm.at[pl.ds(base, chunk)])
    cp2 = pltpu.async_copy(x_hbm.at[idx2_v], buf0, sem0)
    cp1.wait()
    pltpu.sync_copy(buf1, xs_hbm.at[pl.ds(base + chunk, chunk)])
    cp3 = pltpu.async_copy(x_hbm.at[idx3_v], buf1, sem1)
    cp2.wait()
    pltpu.sync_copy(buf0, xs_hbm.at[pl.ds(base + 2 * chunk, chunk)])
    cp3.wait()
    pltpu.sync_copy(buf1, xs_hbm.at[pl.ds(base + 3 * chunk, chunk)])


def _make_sc_gather(total_rows, cols):
    chunk = total_rows // _NW // 4
    return functools.partial(
        pl.kernel,
        mesh=plsc.VectorSubcoreMesh(core_axis_name="c", subcore_axis_name="s"),
        out_type=jax.ShapeDtypeStruct((total_rows, cols), jnp.float32),
        scratch_types=[
            pltpu.VMEM((chunk,), jnp.int32),
            pltpu.VMEM((chunk,), jnp.int32),
            pltpu.VMEM((chunk,), jnp.int32),
            pltpu.VMEM((chunk,), jnp.int32),
            pltpu.VMEM((chunk, cols), jnp.float32),
            pltpu.VMEM((chunk, cols), jnp.float32),
            pltpu.SemaphoreType.DMA,
            pltpu.SemaphoreType.DMA,
        ],
    )(functools.partial(_sc_gather_body, total_rows, cols))


def _gemm_body(be_ref, wsc_ref, xs_ref, wg_ref, wu_ref, wd_ref, ys_ref,
               wg_s, wu_s, wd_s):
    g = pl.program_id(0)
    prev = jnp.where(g == 0, -1, be_ref[jnp.maximum(g - 1, 0)])

    @pl.when(be_ref[g] != prev)
    def _():
        wg_s[...] = wg_ref[0].astype(jnp.bfloat16)
        wu_s[...] = wu_ref[0].astype(jnp.bfloat16)
        wd_s[...] = wd_ref[0].astype(jnp.bfloat16)

    xlo, xhi = _unpack_pairs(xs_ref[...])
    gg = (lax.dot_general(xlo, wg_s[:, :_DH], _DN,
                          preferred_element_type=jnp.float32)
          + lax.dot_general(xhi, wg_s[:, _DH:], _DN,
                            preferred_element_type=jnp.float32))
    uu = (lax.dot_general(xlo, wu_s[:, :_DH], _DN,
                          preferred_element_type=jnp.float32)
          + lax.dot_general(xhi, wu_s[:, _DH:], _DN,
                            preferred_element_type=jnp.float32))
    h = ((gg * jax.nn.sigmoid(gg)) * uu
         * wsc_ref[0, 0, :][:, None]).astype(jnp.bfloat16)
    y = lax.dot_general(h, wd_s[...], _DN,
                        preferred_element_type=jnp.float32)
    ys_ref[...] = _pack_pairs(y)


def _shared_body(x_ref, sg_ref, su_ref, sd_ref, ya_ref, yb_ref, out_ref,
                 sg_s, su_s, sd_s):
    @pl.when(pl.program_id(0) == 0)
    def _():
        sg_s[...] = sg_ref[...].astype(jnp.bfloat16)
        su_s[...] = su_ref[...].astype(jnp.bfloat16)
        sd_s[...] = sd_ref[...].astype(jnp.bfloat16)

    xlo, xhi = _unpack_pairs(x_ref[...])
    g = (lax.dot_general(xlo, sg_s[:, :_DH], _DN,
                         preferred_element_type=jnp.float32)
         + lax.dot_general(xhi, sg_s[:, _DH:], _DN,
                           preferred_element_type=jnp.float32))
    u = (lax.dot_general(xlo, su_s[:, :_DH], _DN,
                         preferred_element_type=jnp.float32)
         + lax.dot_general(xhi, su_s[:, _DH:], _DN,
                           preferred_element_type=jnp.float32))
    h = ((g * jax.nn.sigmoid(g)) * u).astype(jnp.bfloat16)
    shared = lax.dot_general(h, sd_s[...], _DN,
                             preferred_element_type=jnp.float32)
    alo, ahi = _unpack_pairs(ya_ref[...])
    blo, bhi = _unpack_pairs(yb_ref[...])
    out_ref[:, :_DH] = (shared[:, :_DH] + alo.astype(jnp.float32)
                        + blo.astype(jnp.float32))
    out_ref[:, _DH:] = (shared[:, _DH:] + ahi.astype(jnp.float32)
                        + bhi.astype(jnp.float32))


def kernel(hidden_states, router_w, router_bias, Wg, Wu, Wd, Sg, Su, Sd):
    orig_shape = hidden_states.shape
    x = hidden_states.reshape(-1, _D)

    pp, topk_w, cnt, xpk = pl.pallas_call(
        _router_body,
        out_shape=(jax.ShapeDtypeStruct((_T, _K), jnp.int32),
                   jax.ShapeDtypeStruct((_T, _K), jnp.float32),
                   jax.ShapeDtypeStruct((1, _E), jnp.int32),
                   jax.ShapeDtypeStruct((_T, _DH), jnp.float32)),
    )(x, router_w, router_bias.reshape(1, _E))

    counts = cnt.reshape(_E)
    psize = ((counts + (_B - 1)) // _B) * _B
    poff = jnp.concatenate(
        [jnp.zeros(1, jnp.int32), jnp.cumsum(psize).astype(jnp.int32)])
    pp_flat = pp.reshape(-1)
    init = jnp.stack([jnp.arange(_NP, dtype=jnp.int32) % _T,
                      jnp.zeros(_NP, jnp.int32)])
    upd = jnp.stack([jnp.arange(_N, dtype=jnp.int32) // _K,
                     lax.bitcast_convert_type(topk_w.reshape(-1), jnp.int32)])
    merged = init.at[:, pp_flat].set(upd)
    tok_pad = merged[0]
    w_pad = lax.bitcast_convert_type(merged[1], jnp.float32)
    block_expert = jnp.clip(
        jnp.searchsorted(poff, jnp.arange(_NBP, dtype=jnp.int32) * _B,
                         side='right').astype(jnp.int32) - 1,
        0, _E - 1)
    inv_cat = jnp.concatenate([pp[:, 0], pp[:, 1]])

    xs = _make_sc_gather(_NP, _DH)(xpk, tok_pad)

    ys = pl.pallas_call(
        _gemm_body,
        grid_spec=pltpu.PrefetchScalarGridSpec(
            num_scalar_prefetch=1,
            grid=(_NBP,),
            in_specs=[
                pl.BlockSpec((1, 1, _B), lambda g, be: (g, 0, 0)),
                pl.BlockSpec((_B, _DH), lambda g, be: (g, 0)),
                pl.BlockSpec((1, _FF, _D), lambda g, be: (be[g], 0, 0)),
                pl.BlockSpec((1, _FF, _D), lambda g, be: (be[g], 0, 0)),
                pl.BlockSpec((1, _D, _FF), lambda g, be: (be[g], 0, 0)),
            ],
            out_specs=pl.BlockSpec((_B, _DH), lambda g, be: (g, 0)),
            scratch_shapes=[
                pltpu.VMEM((_FF, _D), jnp.bfloat16),
                pltpu.VMEM((_FF, _D), jnp.bfloat16),
                pltpu.VMEM((_D, _FF), jnp.bfloat16),
            ],
        ),
        out_shape=jax.ShapeDtypeStruct((_NP, _DH), jnp.float32),
        compiler_params=pltpu.CompilerParams(
            dimension_semantics=("arbitrary",),
        ),
    )(block_expert, w_pad.reshape(_NBP, 1, _B), xs, Wg, Wu, Wd)

    ysab = _make_sc_gather(2 * _T, _DH)(ys, inv_cat)

    tb = _T // 4
    out = pl.pallas_call(
        _shared_body,
        grid=(4,),
        in_specs=[
            pl.BlockSpec((tb, _DH), lambda t: (t, 0)),
            pl.BlockSpec((_FF, _D), lambda t: (0, 0)),
            pl.BlockSpec((_FF, _D), lambda t: (0, 0)),
            pl.BlockSpec((_D, _FF), lambda t: (0, 0)),
            pl.BlockSpec((tb, _DH), lambda t: (t, 0)),
            pl.BlockSpec((tb, _DH), lambda t: (t + _T // tb, 0)),
        ],
        out_specs=pl.BlockSpec((tb, _D), lambda t: (t, 0)),
        out_shape=jax.ShapeDtypeStruct((_T, _D), jnp.float32),
        scratch_shapes=[
            pltpu.VMEM((_FF, _D), jnp.bfloat16),
            pltpu.VMEM((_FF, _D), jnp.bfloat16),
            pltpu.VMEM((_D, _FF), jnp.bfloat16),
        ],
    )(xpk, Sg, Su, Sd, ysab, ysab)

    return out.reshape(orig_shape)
